# baseline (speedup 1.0000x reference)
import functools

import jax
import jax.numpy as jnp
from jax import lax
from jax.experimental import pallas as pl
from jax.experimental.pallas import tpu as pltpu
from jax.experimental.pallas import tpu_sc as plsc

N, E, D = 10000, 160000, 256
H = D // 2
NC, NS = 2, 16
NP = 10112
RPT = NP // NS
CH = 128
NCHUNK = 80
EPT = NCHUNK * CH
EPAD = NS * EPT
DST_PAD = N + 8
NB = 2
HC = NCHUNK // 2
EPS = 1e-5


def _bn0_body(x_ref, g_ref, b_ref, o_ref):
    x = x_ref[...]
    mean = jnp.mean(x, axis=0, keepdims=True)
    var = jnp.mean((x - mean) ** 2, axis=0, keepdims=True)
    y = (x - mean) / jnp.sqrt(var + EPS) * g_ref[...] + b_ref[...]
    o_ref[:N, :] = y[:, :H]
    o_ref[NP:NP + N, :] = y[:, H:]


def _bn0(x, g, b):
    return pl.pallas_call(
        _bn0_body,
        out_shape=jax.ShapeDtypeStruct((2 * NP, H), jnp.float32),
    )(x, g, b)


def _mlp_body(h_ref, wa_ref, ba_ref, g_ref, b_ref, wb_ref, bb_ref, o_ref,
              *, split_out):
    t = (jnp.dot(h_ref[:N, :], wa_ref[:H, :],
                 preferred_element_type=jnp.float32)
         + jnp.dot(h_ref[NP:NP + N, :], wa_ref[H:, :],
                   preferred_element_type=jnp.float32)
         + ba_ref[...])
    mean = jnp.mean(t, axis=0, keepdims=True)
    var = jnp.mean((t - mean) ** 2, axis=0, keepdims=True)
    t = (t - mean) / jnp.sqrt(var + EPS) * g_ref[...] + b_ref[...]
    t = jnp.maximum(t, 0.0)
    t = jnp.dot(t, wb_ref[...], preferred_element_type=jnp.float32) + bb_ref[...]
    t = jnp.maximum(t, 0.0)
    if split_out:
        o_ref[:N, :] = t[:, :H]
        o_ref[NP:NP + N, :] = t[:, H:]
    else:
        o_ref[...] = t


def _mlp(h_split, wa, ba, g, b, wb, bb, split_out):
    out_shape = (jax.ShapeDtypeStruct((2 * NP, H), jnp.float32) if split_out
                 else jax.ShapeDtypeStruct((N, D), jnp.float32))
    return pl.pallas_call(
        functools.partial(_mlp_body, split_out=split_out),
        out_shape=out_shape,
    )(h_split, wa, ba, g, b, wb, bb)


def _sc_agg_body(xs_hbm, xsf_hbm, src2d_hbm, dst2d_hbm, out_hbm,
                 sbuf, dbuf, r0, r1, g0, g1, s0, s1):
    rows = [r0, r1]
    gsem = [g0, g1]
    ssem = [s0, s1]
    c = lax.axis_index("c")
    s = lax.axis_index("s")
    base = s * RPT
    plsc.subcore_barrier()

    for hv in range(2):
        pltpu.sync_copy(
            src2d_hbm.at[pl.ds((c * NS + s) * NCHUNK + hv * HC, HC)], sbuf)
        pltpu.sync_copy(dst2d_hbm.at[pl.ds(s * NCHUNK + hv * HC, HC)], dbuf)
        pltpu.async_copy(xsf_hbm.at[sbuf.at[0]], rows[0], gsem[0])

        def outer(j0, carry):
            for b in range(NB):
                j = j0 * NB + b
                pltpu.make_async_copy(xsf_hbm.at[sbuf.at[j]], rows[b],
                                      gsem[b]).wait()

                jn = j + 1
                bp = (b + 1) % NB

                @pl.when(jn < HC)
                def _prefetch():
                    pltpu.async_copy(xsf_hbm.at[sbuf.at[jn]], rows[bp],
                                     gsem[bp])
            return carry

        lax.fori_loop(0, HC // NB, outer, 0, unroll=False)

    plsc.subcore_barrier()


_sc_agg = functools.partial(
    pl.kernel,
    out_type=jax.ShapeDtypeStruct((2 * NP, H), jnp.float32),
    mesh=plsc.VectorSubcoreMesh(core_axis_name="c", subcore_axis_name="s"),
    scratch_types=(
        [pltpu.VMEM((HC, CH), jnp.int32),
         pltpu.VMEM((HC, CH), jnp.int32)]
        + [pltpu.VMEM((CH, 2 * H), jnp.float32)] * NB
        + [pltpu.SemaphoreType.DMA] * (2 * NB)
    ),
)(_sc_agg_body)


def kernel(x, adj_t, bn0_g, bn0_b, W1a, b1a, bn1_g, bn1_b, W1b, b1b,
           W2a, b2a, bn2_g, bn2_b, W2b, b2b):
    src = adj_t[0]
    dst = adj_t[1]
    pad = EPAD - E
    src_p = jnp.concatenate([src, jnp.zeros((pad,), jnp.int32)])
    dst_p = jnp.concatenate([dst, jnp.full((pad,), DST_PAD, jnp.int32)])
    src2 = jnp.concatenate([src_p, src_p]).reshape(2 * NS * NCHUNK, CH)
    dst2 = dst_p.reshape(NS * NCHUNK, CH)

    g0 = bn0_g.reshape(1, D)
    b0 = bn0_b.reshape(1, D)
    x_bn = _bn0(x, g0, b0)

    xf1 = jnp.concatenate([x_bn[:NP], x_bn[NP:]], axis=1)
    h1 = _sc_agg(x_bn, xf1, src2, dst2)
    x2 = _mlp(h1, W1a, b1a.reshape(1, D), bn1_g.reshape(1, D),
              bn1_b.reshape(1, D), W1b, b1b.reshape(1, D), split_out=True)

    xf2 = jnp.concatenate([x2[:NP], x2[NP:]], axis=1)
    h2 = _sc_agg(x2, xf2, src2, dst2)
    out = _mlp(h2, W2a, b2a.reshape(1, D), bn2_g.reshape(1, D),
               bn2_b.reshape(1, D), W2b, b2b.reshape(1, D), split_out=False)
    return out


# speedup vs baseline: 1.5850x; 1.5850x over previous
import functools

import jax
import jax.numpy as jnp
from jax import lax
from jax.experimental import pallas as pl
from jax.experimental.pallas import tpu as pltpu
from jax.experimental.pallas import tpu_sc as plsc

N, E, D = 10000, 160000, 256
H = D // 2
NC, NS = 2, 16
NP = 10112
RPT = NP // NS
CH = 128
NCHUNK = 80
EPT = NCHUNK * CH
EPAD = NS * EPT
DST_PAD = N + 8
NB = 2
HC = NCHUNK // 2
EPS = 1e-5


def _bn0_body(x_ref, g_ref, b_ref, o_ref):
    x = x_ref[...]
    mean = jnp.mean(x, axis=0, keepdims=True)
    var = jnp.mean((x - mean) ** 2, axis=0, keepdims=True)
    y = (x - mean) / jnp.sqrt(var + EPS) * g_ref[...] + b_ref[...]
    o_ref[:N, :] = y[:, :H]
    o_ref[NP:NP + N, :] = y[:, H:]


def _bn0(x, g, b):
    return pl.pallas_call(
        _bn0_body,
        out_shape=jax.ShapeDtypeStruct((2 * NP, H), jnp.float32),
    )(x, g, b)


def _mlp_body(h_ref, wa_ref, ba_ref, g_ref, b_ref, wb_ref, bb_ref, o_ref,
              *, split_out):
    t = (jnp.dot(h_ref[:N, :], wa_ref[:H, :],
                 preferred_element_type=jnp.float32)
         + jnp.dot(h_ref[NP:NP + N, :], wa_ref[H:, :],
                   preferred_element_type=jnp.float32)
         + ba_ref[...])
    mean = jnp.mean(t, axis=0, keepdims=True)
    var = jnp.mean((t - mean) ** 2, axis=0, keepdims=True)
    t = (t - mean) / jnp.sqrt(var + EPS) * g_ref[...] + b_ref[...]
    t = jnp.maximum(t, 0.0)
    t = jnp.dot(t, wb_ref[...], preferred_element_type=jnp.float32) + bb_ref[...]
    t = jnp.maximum(t, 0.0)
    if split_out:
        o_ref[:N, :] = t[:, :H]
        o_ref[NP:NP + N, :] = t[:, H:]
    else:
        o_ref[...] = t


def _mlp(h_split, wa, ba, g, b, wb, bb, split_out):
    out_shape = (jax.ShapeDtypeStruct((2 * NP, H), jnp.float32) if split_out
                 else jax.ShapeDtypeStruct((N, D), jnp.float32))
    return pl.pallas_call(
        functools.partial(_mlp_body, split_out=split_out),
        out_shape=out_shape,
    )(h_split, wa, ba, g, b, wb, bb)


def _sc_agg_body(xs_hbm, src2d_hbm, dst2d_hbm, out_hbm,
                 sbuf, dbuf, r0, r1, g0, g1, h0, h1, s0, s1, acc):
    rows = [r0, r1]
    gsem = [g0, g1]
    hsem = [h0, h1]
    ssem = [s0, s1]
    c = lax.axis_index("c")
    s = lax.axis_index("s")
    base = s * RPT
    pltpu.sync_copy(xs_hbm.at[pl.ds(c * NP + base, RPT)],
                    acc.at[pl.ds(base, RPT)])
    plsc.subcore_barrier()

    for hv in range(2):
        pltpu.sync_copy(
            src2d_hbm.at[pl.ds((c * NS + s) * NCHUNK + hv * HC, HC)], sbuf)
        pltpu.sync_copy(dst2d_hbm.at[pl.ds(s * NCHUNK + hv * HC, HC)], dbuf)
        pltpu.async_copy(xs_hbm.at[sbuf.at[0, pl.ds(0, 64)]],
                         rows[0].at[pl.ds(0, 64)], gsem[0])
        pltpu.async_copy(xs_hbm.at[sbuf.at[0, pl.ds(64, 64)]],
                         rows[0].at[pl.ds(64, 64)], hsem[0])

        def outer(j0, carry):
            for b in range(NB):
                j = j0 * NB + b
                pltpu.make_async_copy(xs_hbm.at[sbuf.at[j, pl.ds(0, 64)]],
                                      rows[b].at[pl.ds(0, 64)],
                                      gsem[b]).wait()
                pltpu.make_async_copy(xs_hbm.at[sbuf.at[j, pl.ds(64, 64)]],
                                      rows[b].at[pl.ds(64, 64)],
                                      hsem[b]).wait()
                pltpu.async_copy(rows[b], acc.at[dbuf.at[j]], ssem[b],
                                 add=True)
                jn = j + 1
                bp = (b + 1) % NB

                @pl.when(jn < HC)
                def _prefetch():
                    @pl.when(j >= 1)
                    def _wait_prev():
                        pltpu.make_async_copy(rows[bp], acc.at[dbuf.at[j]],
                                              ssem[bp]).wait()
                    pltpu.async_copy(xs_hbm.at[sbuf.at[jn, pl.ds(0, 64)]],
                                     rows[bp].at[pl.ds(0, 64)], gsem[bp])
                    pltpu.async_copy(xs_hbm.at[sbuf.at[jn, pl.ds(64, 64)]],
                                     rows[bp].at[pl.ds(64, 64)], hsem[bp])
            return carry

        lax.fori_loop(0, HC // NB, outer, 0, unroll=False)
        for b in range(NB):
            pltpu.make_async_copy(rows[b], acc.at[dbuf.at[b]], ssem[b]).wait()

    plsc.subcore_barrier()
    pltpu.sync_copy(acc.at[pl.ds(base, RPT)],
                    out_hbm.at[pl.ds(c * NP + base, RPT)])


_sc_agg = functools.partial(
    pl.kernel,
    out_type=jax.ShapeDtypeStruct((2 * NP, H), jnp.float32),
    mesh=plsc.VectorSubcoreMesh(core_axis_name="c", subcore_axis_name="s"),
    scratch_types=(
        [pltpu.VMEM((HC, CH), jnp.int32),
         pltpu.VMEM((HC, CH), jnp.int32)]
        + [pltpu.VMEM((CH, H), jnp.float32)] * NB
        + [pltpu.SemaphoreType.DMA] * (3 * NB)
        + [pltpu.VMEM_SHARED((NP, H), jnp.float32)]
    ),
)(_sc_agg_body)


def kernel(x, adj_t, bn0_g, bn0_b, W1a, b1a, bn1_g, bn1_b, W1b, b1b,
           W2a, b2a, bn2_g, bn2_b, W2b, b2b):
    src = adj_t[0]
    dst = adj_t[1]
    pad = EPAD - E
    src_p = jnp.concatenate([src, jnp.zeros((pad,), jnp.int32)])
    dst_p = jnp.concatenate([dst, jnp.full((pad,), DST_PAD, jnp.int32)])
    src2 = jnp.concatenate([src_p, src_p + NP]).reshape(2 * NS * NCHUNK, CH)
    dst2 = dst_p.reshape(NS * NCHUNK, CH)

    g0 = bn0_g.reshape(1, D)
    b0 = bn0_b.reshape(1, D)
    x_bn = _bn0(x, g0, b0)

    h1 = _sc_agg(x_bn, src2, dst2)
    x2 = _mlp(h1, W1a, b1a.reshape(1, D), bn1_g.reshape(1, D),
              bn1_b.reshape(1, D), W1b, b1b.reshape(1, D), split_out=True)

    h2 = _sc_agg(x2, src2, dst2)
    out = _mlp(h2, W2a, b2a.reshape(1, D), bn2_g.reshape(1, D),
               bn2_b.reshape(1, D), W2b, b2b.reshape(1, D), split_out=False)
    return out


# speedup vs baseline: 1.5938x; 1.0055x over previous
import functools

import jax
import jax.numpy as jnp
from jax import lax
from jax.experimental import pallas as pl
from jax.experimental.pallas import tpu as pltpu
from jax.experimental.pallas import tpu_sc as plsc

N, E, D = 10000, 160000, 256
H = D // 2
NC, NS = 2, 16
NP = 10112
RPT = NP // NS
CH = 128
NCHUNK = 80
EPT = NCHUNK * CH
EPAD = NS * EPT
DST_PAD = N + 8
NB = 2
HC = NCHUNK // 2
EPS = 1e-5


def _bn0_body(x_ref, g_ref, b_ref, o_ref):
    x = x_ref[...]
    mean = jnp.mean(x, axis=0, keepdims=True)
    var = jnp.mean((x - mean) ** 2, axis=0, keepdims=True)
    y = (x - mean) / jnp.sqrt(var + EPS) * g_ref[...] + b_ref[...]
    o_ref[:N, :] = y[:, :H]
    o_ref[NP:NP + N, :] = y[:, H:]


def _bn0(x, g, b):
    return pl.pallas_call(
        _bn0_body,
        out_shape=jax.ShapeDtypeStruct((2 * NP, H), jnp.float32),
    )(x, g, b)


def _mlp_body(h_ref, wa_ref, ba_ref, g_ref, b_ref, wb_ref, bb_ref, o_ref,
              *, split_out):
    t = (jnp.dot(h_ref[:N, :], wa_ref[:H, :],
                 preferred_element_type=jnp.float32)
         + jnp.dot(h_ref[NP:NP + N, :], wa_ref[H:, :],
                   preferred_element_type=jnp.float32)
         + ba_ref[...])
    mean = jnp.mean(t, axis=0, keepdims=True)
    var = jnp.mean((t - mean) ** 2, axis=0, keepdims=True)
    t = (t - mean) / jnp.sqrt(var + EPS) * g_ref[...] + b_ref[...]
    t = jnp.maximum(t, 0.0)
    t = jnp.dot(t, wb_ref[...], preferred_element_type=jnp.float32) + bb_ref[...]
    t = jnp.maximum(t, 0.0)
    if split_out:
        o_ref[:N, :] = t[:, :H]
        o_ref[NP:NP + N, :] = t[:, H:]
    else:
        o_ref[...] = t


def _mlp(h_split, wa, ba, g, b, wb, bb, split_out):
    out_shape = (jax.ShapeDtypeStruct((2 * NP, H), jnp.float32) if split_out
                 else jax.ShapeDtypeStruct((N, D), jnp.float32))
    return pl.pallas_call(
        functools.partial(_mlp_body, split_out=split_out),
        out_shape=out_shape,
    )(h_split, wa, ba, g, b, wb, bb)


def _sc_agg_body(xs_hbm, src2d_hbm, dst2d_hbm, out_hbm,
                 sbuf, dbuf, r0, r1, g0, g1, s0, s1, isem, acc):
    rows = [r0, r1]
    gsem = [g0, g1]
    ssem = [s0, s1]
    c = lax.axis_index("c")
    s = lax.axis_index("s")
    base = s * RPT
    pltpu.async_copy(xs_hbm.at[pl.ds(c * NP + base, RPT)],
                     acc.at[pl.ds(base, RPT)], isem)

    for hv in range(2):
        pltpu.sync_copy(
            src2d_hbm.at[pl.ds((c * NS + s) * NCHUNK + hv * HC, HC)], sbuf)
        pltpu.sync_copy(dst2d_hbm.at[pl.ds(s * NCHUNK + hv * HC, HC)], dbuf)
        pltpu.async_copy(xs_hbm.at[sbuf.at[0]], rows[0], gsem[0])
        if hv == 0:
            pltpu.make_async_copy(xs_hbm.at[pl.ds(c * NP + base, RPT)],
                                  acc.at[pl.ds(base, RPT)], isem).wait()
            plsc.subcore_barrier()

        def outer(j0, carry):
            for b in range(NB):
                j = j0 * NB + b
                pltpu.make_async_copy(xs_hbm.at[sbuf.at[j]], rows[b],
                                      gsem[b]).wait()
                pltpu.async_copy(rows[b], acc.at[dbuf.at[j]], ssem[b],
                                 add=True)
                jn = j + 1
                bp = (b + 1) % NB

                @pl.when(jn < HC)
                def _prefetch():
                    @pl.when(j >= 1)
                    def _wait_prev():
                        pltpu.make_async_copy(rows[bp], acc.at[dbuf.at[j]],
                                              ssem[bp]).wait()
                    pltpu.async_copy(xs_hbm.at[sbuf.at[jn]], rows[bp],
                                     gsem[bp])
            return carry

        lax.fori_loop(0, HC // NB, outer, 0, unroll=False)
        for b in range(NB):
            pltpu.make_async_copy(rows[b], acc.at[dbuf.at[b]], ssem[b]).wait()

    plsc.subcore_barrier()
    pltpu.sync_copy(acc.at[pl.ds(base, RPT)],
                    out_hbm.at[pl.ds(c * NP + base, RPT)])


_sc_agg = functools.partial(
    pl.kernel,
    out_type=jax.ShapeDtypeStruct((2 * NP, H), jnp.float32),
    mesh=plsc.VectorSubcoreMesh(core_axis_name="c", subcore_axis_name="s"),
    scratch_types=(
        [pltpu.VMEM((HC, CH), jnp.int32),
         pltpu.VMEM((HC, CH), jnp.int32)]
        + [pltpu.VMEM((CH, H), jnp.float32)] * NB
        + [pltpu.SemaphoreType.DMA] * (2 * NB + 1)
        + [pltpu.VMEM_SHARED((NP, H), jnp.float32)]
    ),
)(_sc_agg_body)


def kernel(x, adj_t, bn0_g, bn0_b, W1a, b1a, bn1_g, bn1_b, W1b, b1b,
           W2a, b2a, bn2_g, bn2_b, W2b, b2b):
    src = adj_t[0]
    dst = adj_t[1]
    pad = EPAD - E
    src_p = jnp.concatenate([src, jnp.zeros((pad,), jnp.int32)])
    dst_p = jnp.concatenate([dst, jnp.full((pad,), DST_PAD, jnp.int32)])
    src2 = jnp.concatenate([src_p, src_p + NP]).reshape(2 * NS * NCHUNK, CH)
    dst2 = dst_p.reshape(NS * NCHUNK, CH)

    g0 = bn0_g.reshape(1, D)
    b0 = bn0_b.reshape(1, D)
    x_bn = _bn0(x, g0, b0)

    h1 = _sc_agg(x_bn, src2, dst2)
    x2 = _mlp(h1, W1a, b1a.reshape(1, D), bn1_g.reshape(1, D),
              bn1_b.reshape(1, D), W1b, b1b.reshape(1, D), split_out=True)

    h2 = _sc_agg(x2, src2, dst2)
    out = _mlp(h2, W2a, b2a.reshape(1, D), bn2_g.reshape(1, D),
               bn2_b.reshape(1, D), W2b, b2b.reshape(1, D), split_out=False)
    return out
